# baseline (speedup 1.0000x reference)
import functools

import jax
import jax.numpy as jnp
from jax import lax
from jax.experimental import pallas as pl
from jax.experimental.pallas import tpu as pltpu
from jax.experimental.pallas import tpu_sc as plsc

_N = 10000
_E = 320000
_D = 128

_CG = 32
_CW = _D // _CG
_GW = 2 * _CW
_B = 1600
_SUB = 80
_NCH = _E // _B
_ACC = _N * _CW


def _matmul_body(x_ref, w_ref, o_ref):
    o_ref[...] = jnp.dot(x_ref[...], w_ref[...],
                         preferred_element_type=jnp.float32)


def _matmul(x, W):
    blk = 1000
    return pl.pallas_call(
        _matmul_body,
        grid=(_N // blk,),
        in_specs=[
            pl.BlockSpec((blk, _D), lambda i: (i, 0)),
            pl.BlockSpec((_D, _D), lambda i: (0, 0)),
        ],
        out_specs=pl.BlockSpec((blk, _D), lambda i: (i, 0)),
        out_shape=jax.ShapeDtypeStruct((_N, _D), jnp.float32),
    )(x, W)


def _sc_scatter_max(normT, row, col):
    mesh = plsc.VectorSubcoreMesh(core_axis_name="c", subcore_axis_name="s")

    @functools.partial(
        pl.kernel,
        mesh=mesh,
        out_type=jax.ShapeDtypeStruct((_CG, _ACC), jnp.float32),
        compiler_params=pltpu.CompilerParams(
            needs_layout_passes=False, use_tc_tiling_on_sc=False),
        scratch_types=(
            [pltpu.VMEM((_ACC,), jnp.float32)] * 2
            + [pltpu.VMEM((_B,), jnp.int32)] * 4
            + [pltpu.VMEM((_B,), jnp.int32)] * 4
            + [pltpu.VMEM((_B, _GW), jnp.float32)] * 2
            + [pltpu.SemaphoreType.DMA] * 6
        ),
    )
    def body(normT_hbm, row_hbm, col_hbm, out_hbm,
             acc0, acc1, r0, r1, r2, r3, c0, c1, c2, c3, g0, g1,
             si0, si1, si2, si3, sg0, sg1):
        c = lax.axis_index("c")
        s = lax.axis_index("s")
        g = c * 16 + s
        table = normT_hbm.at[g >> 1]

        acc = (acc0, acc1)
        ridx = (r0, r1, r2, r3)
        cidx = (c0, c1, c2, c3)
        gbuf = (g0, g1)
        semi = (si0, si1, si2, si3)
        semg = (sg0, sg1)

        neg_inf = jnp.full((16,), -jnp.inf, jnp.float32)

        def init_body(i, carry):
            acc0[pl.ds(i * 16, 16)] = neg_inf
            acc1[pl.ds(i * 16, 16)] = neg_inf
            return carry
        lax.fori_loop(0, _ACC // 16, init_body, 0, unroll=8)

        lane = lax.iota(jnp.int32, 16)
        e4 = lane >> 2
        lo = lane & 3
        loD = lo + (g & 1) * _CW
        rotp = [(lane + 4 * r) & 15 for r in (1, 2, 3)]
        eperm = [4 * q + e4 for q in range(4)]
        qrot = [(lane & 12) | ((lane + r) & 3) for r in (1, 2, 3)]

        def fire_idx(b, ch):
            off = ch * _B
            pltpu.async_copy(row_hbm.at[pl.ds(off, _B)], ridx[b], semi[b])
            pltpu.async_copy(col_hbm.at[pl.ds(off, _B)], cidx[b], semi[b])

        def wait_idx(b, ch):
            off = ch * _B
            pltpu.make_async_copy(
                row_hbm.at[pl.ds(off, _B)], ridx[b], semi[b]).wait()
            pltpu.make_async_copy(
                col_hbm.at[pl.ds(off, _B)], cidx[b], semi[b]).wait()

        def fire_gathers(b):
            pltpu.async_copy(table.at[ridx[b]], gbuf[b & 1], semg[b & 1])

        def wait_gathers(b):
            pltpu.make_async_copy(
                table.at[ridx[b]], gbuf[b & 1], semg[b & 1]).wait()

        def process(b):
            gb = gbuf[b & 1]
            cb = cidx[b]

            def blk_body(i, carry):
                base = 16 * i
                cvec = cb[pl.ds(base, 16)]
                dup = (cvec == jnp.take_along_axis(cvec, qrot[0], axis=0))
                dup |= (cvec == jnp.take_along_axis(cvec, qrot[1], axis=0))
                dup |= (cvec == jnp.take_along_axis(cvec, qrot[2], axis=0))
                has_dup = jnp.any(dup)

                @pl.when(jnp.logical_not(has_dup))
                def _fast():
                    for q in range(4):
                        cols = jnp.take_along_axis(cvec, eperm[q], axis=0)
                        data = plsc.load_gather(gb, [base + eperm[q], loD])
                        addr = cols * _CW + lo
                        a = acc[q & 1]
                        old = plsc.load_gather(a, [addr])
                        plsc.store_scatter(a, [addr],
                                           jnp.maximum(old, data))

                @pl.when(has_dup)
                def _slow():
                    for q in range(4):
                        cols = jnp.take_along_axis(cvec, eperm[q], axis=0)
                        data = plsc.load_gather(gb, [base + eperm[q], loD])
                        for rp in rotp:
                            colsr = jnp.take_along_axis(cols, rp, axis=0)
                            datar = jnp.take_along_axis(data, rp, axis=0)
                            data = jnp.where(cols == colsr,
                                             jnp.maximum(data, datar), data)
                        addr = cols * _CW + lo
                        a = acc[q & 1]
                        old = plsc.load_gather(a, [addr])
                        plsc.store_scatter(a, [addr],
                                           jnp.maximum(old, data))
                return carry
            lax.fori_loop(0, _B // 16, blk_body, 0, unroll=4)

        fire_idx(0, 0)
        fire_idx(1, 1)
        wait_idx(0, 0)
        fire_gathers(0)
        wait_idx(1, 1)
        fire_gathers(1)

        def pipe_body(n, carry):
            for b in range(4):
                ch = 4 * n + b
                nb = (b + 2) % 4
                wait_gathers(b)

                @pl.when(ch + 2 < _NCH)
                def _prefetch_idx():
                    fire_idx(nb, ch + 2)

                process(b)

                @pl.when(ch + 2 < _NCH)
                def _start_gather():
                    wait_idx(nb, ch + 2)
                    fire_gathers(nb)
            return carry
        lax.fori_loop(0, _NCH // 4, pipe_body, 0)

        zero = jnp.zeros((16,), jnp.float32)

        def fix_body(i, carry):
            sl = pl.ds(i * 16, 16)
            v = jnp.maximum(acc0[sl], acc1[sl])
            acc0[sl] = jnp.where(v == -jnp.inf, zero, v)
            return carry
        lax.fori_loop(0, _ACC // 16, fix_body, 0, unroll=8)

        pltpu.sync_copy(acc0, out_hbm.at[g])

    return body(normT, row, col)


def kernel(x, edge_index, W):
    norm = _matmul(x, W)
    normT = norm.reshape(_N, _CG // 2, _GW).transpose(1, 0, 2)
    pooled = _sc_scatter_max(normT, edge_index[0], edge_index[1])
    pooled = pooled.reshape(_CG, _N, _CW).transpose(1, 0, 2).reshape(_N, _D)
    return jnp.concatenate((x, pooled), axis=1)


# speedup vs baseline: 1.0718x; 1.0718x over previous
import functools

import jax
import jax.numpy as jnp
from jax import lax
from jax.experimental import pallas as pl
from jax.experimental.pallas import tpu as pltpu
from jax.experimental.pallas import tpu_sc as plsc

_N = 10000
_E = 320000
_D = 128

_CG = 32
_CW = _D // _CG
_GW = 2 * _CW
_B = 1600
_SUB = 80
_NCH = _E // _B
_ACC = _N * _CW


def _matmul_body(x_ref, w_ref, o_ref):
    o_ref[...] = jnp.dot(x_ref[...], w_ref[...],
                         preferred_element_type=jnp.float32)


def _matmul(x, W):
    blk = 1000
    return pl.pallas_call(
        _matmul_body,
        grid=(_N // blk,),
        in_specs=[
            pl.BlockSpec((blk, _D), lambda i: (i, 0)),
            pl.BlockSpec((_D, _D), lambda i: (0, 0)),
        ],
        out_specs=pl.BlockSpec((blk, _D), lambda i: (i, 0)),
        out_shape=jax.ShapeDtypeStruct((_N, _D), jnp.float32),
    )(x, W)


def _sc_scatter_max(normT, row, col):
    mesh = plsc.VectorSubcoreMesh(core_axis_name="c", subcore_axis_name="s")

    @functools.partial(
        pl.kernel,
        mesh=mesh,
        out_type=jax.ShapeDtypeStruct((_CG, _ACC), jnp.float32),
        compiler_params=pltpu.CompilerParams(
            needs_layout_passes=False, use_tc_tiling_on_sc=False),
        scratch_types=(
            [pltpu.VMEM((_ACC,), jnp.float32)] * 2
            + [pltpu.VMEM((_B,), jnp.int32)] * 4
            + [pltpu.VMEM((_B,), jnp.int32)] * 4
            + [pltpu.VMEM((_B, _GW), jnp.float32)] * 2
            + [pltpu.SemaphoreType.DMA] * 6
        ),
    )
    def body(normT_hbm, row_hbm, col_hbm, out_hbm,
             acc0, acc1, r0, r1, r2, r3, c0, c1, c2, c3, g0, g1,
             si0, si1, si2, si3, sg0, sg1):
        c = lax.axis_index("c")
        s = lax.axis_index("s")
        g = c * 16 + s
        table = normT_hbm.at[g >> 1]

        acc = (acc0, acc1)
        ridx = (r0, r1, r2, r3)
        cidx = (c0, c1, c2, c3)
        gbuf = (g0, g1)
        semi = (si0, si1, si2, si3)
        semg = (sg0, sg1)

        neg_inf = jnp.full((16,), -jnp.inf, jnp.float32)

        def init_body(i, carry):
            acc0[pl.ds(i * 16, 16)] = neg_inf
            acc1[pl.ds(i * 16, 16)] = neg_inf
            return carry
        lax.fori_loop(0, _ACC // 16, init_body, 0, unroll=8)

        lane = lax.iota(jnp.int32, 16)
        e4 = lane >> 2
        lo = lane & 3
        loD = lo + (g & 1) * _CW
        rotp = [(lane + 4 * r) & 15 for r in (1, 2, 3)]
        eperm = [4 * q + e4 for q in range(4)]
        eperm_r = [[4 * q + ((e4 + r) & 3) for r in (1, 2, 3)]
                   for q in range(4)]

        def fire_idx(b, ch):
            off = ch * _B
            pltpu.async_copy(row_hbm.at[pl.ds(off, _B)], ridx[b], semi[b])
            pltpu.async_copy(col_hbm.at[pl.ds(off, _B)], cidx[b], semi[b])

        def wait_idx(b, ch):
            off = ch * _B
            pltpu.make_async_copy(
                row_hbm.at[pl.ds(off, _B)], ridx[b], semi[b]).wait()
            pltpu.make_async_copy(
                col_hbm.at[pl.ds(off, _B)], cidx[b], semi[b]).wait()

        def fire_gathers(b):
            pltpu.async_copy(table.at[ridx[b]], gbuf[b & 1], semg[b & 1])

        def wait_gathers(b):
            pltpu.make_async_copy(
                table.at[ridx[b]], gbuf[b & 1], semg[b & 1]).wait()

        def process(b):
            gb = gbuf[b & 1]
            cb = cidx[b]

            def quad_body(i, carry):
              for q in range(2):
                base = 8 * i + 4 * q
                cols = plsc.load_gather(cb, [base + e4])
                data = plsc.load_gather(gb, [base + e4, loD])
                for r in range(3):
                    colsr = plsc.load_gather(cb, [base + eperm_r[0][r]])
                    datar = jnp.take_along_axis(data, rotp[r], axis=0)
                    data = jnp.where(cols == colsr,
                                     jnp.maximum(data, datar), data)
                addr = cols * _CW + lo
                a = acc[q]
                old = plsc.load_gather(a, [addr])
                plsc.store_scatter(a, [addr], jnp.maximum(old, data))
              return carry
            lax.fori_loop(0, _B // 8, quad_body, 0, unroll=4)

        fire_idx(0, 0)
        fire_idx(1, 1)
        wait_idx(0, 0)
        fire_gathers(0)
        wait_idx(1, 1)
        fire_gathers(1)

        def pipe_body(n, carry):
            for b in range(4):
                ch = 4 * n + b
                nb = (b + 2) % 4
                wait_gathers(b)

                @pl.when(ch + 2 < _NCH)
                def _prefetch_idx():
                    fire_idx(nb, ch + 2)

                process(b)

                @pl.when(ch + 2 < _NCH)
                def _start_gather():
                    wait_idx(nb, ch + 2)
                    fire_gathers(nb)
            return carry
        lax.fori_loop(0, _NCH // 4, pipe_body, 0)

        zero = jnp.zeros((16,), jnp.float32)

        def fix_body(i, carry):
            sl = pl.ds(i * 16, 16)
            v = jnp.maximum(acc0[sl], acc1[sl])
            acc0[sl] = jnp.where(v == -jnp.inf, zero, v)
            return carry
        lax.fori_loop(0, _ACC // 16, fix_body, 0, unroll=8)

        pltpu.sync_copy(acc0, out_hbm.at[g])

    return body(normT, row, col)


def kernel(x, edge_index, W):
    norm = _matmul(x, W)
    normT = norm.reshape(_N, _CG // 2, _GW).transpose(1, 0, 2)
    pooled = _sc_scatter_max(normT, edge_index[0], edge_index[1])
    pooled = pooled.reshape(_CG, _N, _CW).transpose(1, 0, 2).reshape(_N, _D)
    return jnp.concatenate((x, pooled), axis=1)


# speedup vs baseline: 1.0809x; 1.0085x over previous
import functools

import jax
import jax.numpy as jnp
from jax import lax
from jax.experimental import pallas as pl
from jax.experimental.pallas import tpu as pltpu
from jax.experimental.pallas import tpu_sc as plsc

_N = 10000
_E = 320000
_D = 128

_CG = 32
_CW = _D // _CG
_GW = 2 * _CW
_B = 1600
_SUB = 80
_NCH = _E // _B
_ACC = _N * _CW


def _matmul_body(x_ref, w_ref, o_ref):
    o_ref[...] = jnp.dot(x_ref[...], w_ref[...],
                         preferred_element_type=jnp.float32)


def _matmul(x, W):
    blk = 1000
    return pl.pallas_call(
        _matmul_body,
        grid=(_N // blk,),
        in_specs=[
            pl.BlockSpec((blk, _D), lambda i: (i, 0)),
            pl.BlockSpec((_D, _D), lambda i: (0, 0)),
        ],
        out_specs=pl.BlockSpec((blk, _D), lambda i: (i, 0)),
        out_shape=jax.ShapeDtypeStruct((_N, _D), jnp.float32),
    )(x, W)


def _sc_scatter_max(normT, row, col):
    mesh = plsc.VectorSubcoreMesh(core_axis_name="c", subcore_axis_name="s")

    @functools.partial(
        pl.kernel,
        mesh=mesh,
        out_type=jax.ShapeDtypeStruct((_CG, _ACC), jnp.float32),
        compiler_params=pltpu.CompilerParams(
            needs_layout_passes=False, use_tc_tiling_on_sc=False),
        scratch_types=(
            [pltpu.VMEM((_ACC,), jnp.float32)] * 2
            + [pltpu.VMEM((_B,), jnp.int32)] * 4
            + [pltpu.VMEM((_B,), jnp.int32)] * 4
            + [pltpu.VMEM((_B, _GW), jnp.float32)] * 2
            + [pltpu.SemaphoreType.DMA] * 6
        ),
    )
    def body(normT_hbm, row_hbm, col_hbm, out_hbm,
             acc0, acc1, r0, r1, r2, r3, c0, c1, c2, c3, g0, g1,
             si0, si1, si2, si3, sg0, sg1):
        c = lax.axis_index("c")
        s = lax.axis_index("s")
        g = c * 16 + s
        table = normT_hbm.at[g >> 1]

        acc = (acc0, acc1)
        ridx = (r0, r1, r2, r3)
        cidx = (c0, c1, c2, c3)
        gbuf = (g0, g1)
        semi = (si0, si1, si2, si3)
        semg = (sg0, sg1)

        neg_inf = jnp.full((16,), -jnp.inf, jnp.float32)

        def init_body(i, carry):
            acc0[pl.ds(i * 16, 16)] = neg_inf
            acc1[pl.ds(i * 16, 16)] = neg_inf
            return carry
        lax.fori_loop(0, _ACC // 16, init_body, 0, unroll=8)

        lane = lax.iota(jnp.int32, 16)
        e4 = lane >> 2
        lo = lane & 3
        loD = lo + (g & 1) * _CW
        rotp = [(lane + 4 * r) & 15 for r in (1, 2, 3)]
        eperm = [4 * q + e4 for q in range(4)]

        def fire_idx(b, ch):
            off = ch * _B
            pltpu.async_copy(row_hbm.at[pl.ds(off, _B)], ridx[b], semi[b])
            pltpu.async_copy(col_hbm.at[pl.ds(off, _B)], cidx[b], semi[b])

        def wait_idx(b, ch):
            off = ch * _B
            pltpu.make_async_copy(
                row_hbm.at[pl.ds(off, _B)], ridx[b], semi[b]).wait()
            pltpu.make_async_copy(
                col_hbm.at[pl.ds(off, _B)], cidx[b], semi[b]).wait()

        def fire_gathers(b):
            pltpu.async_copy(table.at[ridx[b]], gbuf[b & 1], semg[b & 1])

        def wait_gathers(b):
            pltpu.make_async_copy(
                table.at[ridx[b]], gbuf[b & 1], semg[b & 1]).wait()

        def process(b):
            gb = gbuf[b & 1]
            cb = cidx[b]

            def blk_body(i, carry):
                base = 16 * i
                cvec = cb[pl.ds(base, 16)]
                for q in range(4):
                    cols = jnp.take_along_axis(cvec, eperm[q], axis=0)
                    data = plsc.load_gather(gb, [base + eperm[q], loD])
                    for rp in rotp:
                        colsr = jnp.take_along_axis(cols, rp, axis=0)
                        datar = jnp.take_along_axis(data, rp, axis=0)
                        data = jnp.where(cols == colsr,
                                         jnp.maximum(data, datar), data)
                    addr = cols * _CW + lo
                    a = acc[q & 1]
                    old = plsc.load_gather(a, [addr])
                    plsc.store_scatter(a, [addr], jnp.maximum(old, data))
                return carry
            lax.fori_loop(0, _B // 16, blk_body, 0, unroll=4)

        fire_idx(0, 0)
        fire_idx(1, 1)
        wait_idx(0, 0)
        fire_gathers(0)
        wait_idx(1, 1)
        fire_gathers(1)

        def pipe_body(n, carry):
            for b in range(4):
                ch = 4 * n + b
                nb = (b + 2) % 4
                wait_gathers(b)

                @pl.when(ch + 2 < _NCH)
                def _prefetch_idx():
                    fire_idx(nb, ch + 2)

                process(b)

                @pl.when(ch + 2 < _NCH)
                def _start_gather():
                    wait_idx(nb, ch + 2)
                    fire_gathers(nb)
            return carry
        lax.fori_loop(0, _NCH // 4, pipe_body, 0)

        zero = jnp.zeros((16,), jnp.float32)

        def fix_body(i, carry):
            sl = pl.ds(i * 16, 16)
            v = jnp.maximum(acc0[sl], acc1[sl])
            acc0[sl] = jnp.where(v == -jnp.inf, zero, v)
            return carry
        lax.fori_loop(0, _ACC // 16, fix_body, 0, unroll=8)

        pltpu.sync_copy(acc0, out_hbm.at[g])

    return body(normT, row, col)


def kernel(x, edge_index, W):
    norm = _matmul(x, W)
    normT = norm.reshape(_N, _CG // 2, _GW).transpose(1, 0, 2)
    pooled = _sc_scatter_max(normT, edge_index[0], edge_index[1])
    pooled = pooled.reshape(_CG, _N, _CW).transpose(1, 0, 2).reshape(_N, _D)
    return jnp.concatenate((x, pooled), axis=1)
